# baseline (speedup 1.0000x reference)
import functools

import jax
import jax.numpy as jnp
from jax import lax
from jax.experimental import pallas as pl
from jax.experimental.pallas import tpu as pltpu
from jax.experimental.pallas import tpu_sc as plsc

NC = 2
NS = 16
L = 16
NW = NC * NS
CHUNK = 64
NBUF = 2
SPLIT0 = 0.60


def _pre_body(logits_ref, nf_ref, wp_ref, bp_ref, ex_ref, hv_ref):
    l = logits_ref[...]
    m = jnp.max(l)
    ex_ref[...] = jnp.exp(l - m)
    hv_ref[...] = (
        jnp.dot(nf_ref[...], wp_ref[...], preferred_element_type=jnp.float32)
        + bp_ref[...]
    )


def _make_agg(v_pad, kc0, kc1):
    rps = v_pad // NS
    kcmax = max(kc0, kc1)

    mesh = plsc.VectorSubcoreMesh(core_axis_name="c", subcore_axis_name="s")

    @functools.partial(
        pl.kernel,
        mesh=mesh,
        out_type=[
            jax.ShapeDtypeStruct((NC, v_pad, 128), jnp.float32),
            jax.ShapeDtypeStruct((NC, v_pad), jnp.float32),
        ],
        scratch_types=[
            pltpu.VMEM((kcmax, CHUNK), jnp.int32),
            [pltpu.VMEM((CHUNK,), jnp.int32) for _ in range(NBUF)],
            [pltpu.VMEM((CHUNK,), jnp.float32) for _ in range(NBUF)],
            [pltpu.VMEM((CHUNK, 128), jnp.float32) for _ in range(NBUF)],
            pltpu.VMEM((640,), jnp.float32),
            pltpu.VMEM_SHARED((v_pad, 128), jnp.float32),
            pltpu.VMEM_SHARED((v_pad,), jnp.float32),
            [pltpu.SemaphoreType.DMA for _ in range(NBUF)],
            [pltpu.SemaphoreType.DMA for _ in range(NBUF)],
        ],
    )
    def agg(hv_hbm, src_hbm, dst_hbm, ex_hbm, cm_out, s_out,
            src_v, dst_b, ex_b, rows_b, zs_v, cm_sh, s_sh, isem, osem):
        c = lax.axis_index("c")
        s = lax.axis_index("s")
        base = pl.multiple_of(
            jnp.where(c == 0, NS * kc1 + s * kc0, s * kc1), 8)
        nch = jnp.where(c == 0, kc0, kc1)

        zv = jnp.zeros((L,), jnp.float32)

        def zrow(i, carry):
            for j in range(128 // L):
                rows_b[0][i, pl.ds(j * L, L)] = zv
            return carry

        lax.fori_loop(0, CHUNK, zrow, 0)

        def zsrow(i, carry):
            zs_v[pl.ds(i * L, L)] = zv
            return carry

        lax.fori_loop(0, 640 // L, zsrow, 0)

        nfull = rps // CHUNK
        for i in range(nfull):
            pltpu.sync_copy(rows_b[0],
                            cm_sh.at[pl.ds(s * rps + i * CHUNK, CHUNK)])
        rem = rps - nfull * CHUNK
        if rem:
            pltpu.sync_copy(rows_b[0].at[pl.ds(0, rem)],
                            cm_sh.at[pl.ds(s * rps + nfull * CHUNK, rem)])
        pltpu.sync_copy(zs_v.at[pl.ds(0, rps)] if rps != 640 else zs_v,
                        s_sh.at[pl.ds(s * rps, rps)])

        pltpu.sync_copy(src_hbm.at[pl.ds(base, kcmax)], src_v)

        plsc.subcore_barrier()

        def issue_in(q, b):
            pltpu.async_copy(hv_hbm.at[src_v.at[q]], rows_b[b], isem[b])
            pltpu.async_copy(dst_hbm.at[base + q], dst_b[b], isem[b])
            pltpu.async_copy(ex_hbm.at[base + q], ex_b[b], isem[b])

        def wait_in(q, b):
            pltpu.make_async_copy(hv_hbm.at[src_v.at[q]], rows_b[b],
                                  isem[b]).wait()
            pltpu.make_async_copy(dst_hbm.at[base + q], dst_b[b],
                                  isem[b]).wait()
            pltpu.make_async_copy(ex_hbm.at[base + q], ex_b[b],
                                  isem[b]).wait()

        def issue_out(b):
            pltpu.async_copy(rows_b[b], cm_sh.at[dst_b[b]], osem[b], add=True)
            pltpu.async_copy(ex_b[b], s_sh.at[dst_b[b]], osem[b], add=True)

        def wait_out(b):
            pltpu.make_async_copy(rows_b[b], cm_sh.at[dst_b[b]],
                                  osem[b]).wait()
            pltpu.make_async_copy(ex_b[b], s_sh.at[dst_b[b]], osem[b]).wait()

        issue_in(0, 0)

        nkk = nch // NBUF
        kk_last = nkk - 1

        def outer(kk, carry):
            k0 = kk * NBUF
            for b in range(NBUF):
                k = k0 + b
                ob = 1 - b
                wait_in(k, b)

                if b == 0:
                    @pl.when(kk > 0)
                    def _():
                        wait_out(ob)

                    issue_in(k + 1, ob)
                else:
                    @pl.when(kk < kk_last)
                    def _():
                        wait_out(ob)
                        issue_in(k + 1, ob)

                def scale_g(g, carry2, _b=b):
                    gb = g * L
                    evec = ex_b[_b][pl.ds(gb, L)]
                    for i in range(L):
                        e = evec[i]
                        for jj in range(128 // L):
                            sl = pl.ds(jj * L, L)
                            rows_b[_b][gb + i, sl] = rows_b[_b][gb + i, sl] * e
                    return carry2

                lax.fori_loop(0, CHUNK // L, scale_g, 0)

                issue_out(b)
            return carry

        lax.fori_loop(0, nkk, outer, 0)

        for b in range(NBUF):
            wait_out(b)

        plsc.subcore_barrier()

        pltpu.sync_copy(cm_sh.at[pl.ds(s * rps, rps)],
                        cm_out.at[c].at[pl.ds(s * rps, rps)])
        pltpu.sync_copy(s_sh.at[pl.ds(s * rps, rps)],
                        s_out.at[c].at[pl.ds(s * rps, rps)])

    return agg


def _gru_body(cm_ref, s_ref, nf_ref, wih_ref, bih_ref, whh_ref, bhh_ref, o_ref):
    cm = cm_ref[0] + cm_ref[1]
    sv = s_ref[0] + s_ref[1]
    pos = sv > 0.0
    ctx = jnp.where(pos, cm / jnp.where(pos, sv, 1.0), 0.0)
    ctx = jnp.where(ctx > 0.0, ctx, jnp.exp(ctx) - 1.0)
    nf = nf_ref[...]
    gi = jnp.dot(ctx, wih_ref[...], preferred_element_type=jnp.float32) + bih_ref[...]
    gh = jnp.dot(nf, whh_ref[...], preferred_element_type=jnp.float32) + bhh_ref[...]
    d = o_ref.shape[1]
    r = jax.nn.sigmoid(gi[:, :d] + gh[:, :d])
    z = jax.nn.sigmoid(gi[:, d:2 * d] + gh[:, d:2 * d])
    n = jnp.tanh(gi[:, 2 * d:] + r * gh[:, 2 * d:])
    o_ref[...] = jnp.maximum((1.0 - z) * n + z * nf, 0.0)


def kernel(edge_index, edge_logits, node_feats, W_proj, b_proj, W_ih, b_ih,
           W_hh, b_hh):
    V, D = node_feats.shape
    H = W_proj.shape[0]
    E = edge_index.shape[1]

    ew = NS * CHUNK * 16
    e_pad = -(-E // ew) * ew
    kcol = e_pad // (NS * CHUNK)
    kc0 = min(kcol - 8, int(round(kcol * SPLIT0 / 8)) * 8)
    kc1 = kcol - kc0
    kcmax = max(kc0, kc1)
    ncht = e_pad // CHUNK

    rps = -(-V // (NS * 128)) * 128
    v_pad = NS * rps

    def pad_flat(flat, fill):
        return jnp.concatenate(
            [flat, jnp.full((e_pad - E,), fill, flat.dtype)]
        ).reshape(ncht, CHUNK)

    src = pad_flat(edge_index[0].astype(jnp.int32), 0)
    dst = pad_flat(edge_index[1].astype(jnp.int32), 0)

    logits = jnp.concatenate(
        [edge_logits[:, 0], jnp.full((e_pad - E,), -jnp.inf, jnp.float32)]
    ).reshape(e_pad // 128, 128)

    ex, hv = pl.pallas_call(
        _pre_body,
        out_shape=[
            jax.ShapeDtypeStruct((e_pad // 128, 128), jnp.float32),
            jax.ShapeDtypeStruct((V, H), jnp.float32),
        ],
    )(logits, node_feats, W_proj.T, b_proj.reshape(1, H))

    ex_blk = ex.reshape(ncht, CHUNK)

    agg = _make_agg(v_pad, kc0, kc1)
    cm, sacc = agg(hv, src, dst, ex_blk)

    nf_pad = jnp.concatenate(
        [node_feats, jnp.zeros((v_pad - V, D), jnp.float32)]
    )
    bv = rps
    grid = v_pad // bv
    out = pl.pallas_call(
        _gru_body,
        grid=(grid,),
        in_specs=[
            pl.BlockSpec((NC, bv, H), lambda i: (0, i, 0)),
            pl.BlockSpec((NC, bv, 1), lambda i: (0, i, 0)),
            pl.BlockSpec((bv, D), lambda i: (i, 0)),
            pl.BlockSpec((H, 3 * D), lambda i: (0, 0)),
            pl.BlockSpec((1, 3 * D), lambda i: (0, 0)),
            pl.BlockSpec((D, 3 * D), lambda i: (0, 0)),
            pl.BlockSpec((1, 3 * D), lambda i: (0, 0)),
        ],
        out_specs=pl.BlockSpec((bv, D), lambda i: (i, 0)),
        out_shape=jax.ShapeDtypeStruct((v_pad, D), jnp.float32),
    )(cm, sacc.reshape(NC, v_pad, 1), nf_pad, W_ih.T, b_ih.reshape(1, 3 * D),
      W_hh.T, b_hh.reshape(1, 3 * D))

    return out[:V]


# speedup vs baseline: 1.8737x; 1.8737x over previous
import functools

import jax
import jax.numpy as jnp
from jax import lax
from jax.experimental import pallas as pl
from jax.experimental.pallas import tpu as pltpu
from jax.experimental.pallas import tpu_sc as plsc

NC = 2
NS = 16
L = 16
NW = NC * NS
CHUNK = 64
NBUF = 2
SPLIT0 = 0.60


def _pre_body(logits_ref, nf_ref, wp_ref, bp_ref, ex_ref, hv_ref):
    l = logits_ref[...]
    m = jnp.max(l)
    ex_ref[...] = jnp.exp(l - m)
    hv_ref[...] = (
        jnp.dot(nf_ref[...], wp_ref[...], preferred_element_type=jnp.float32)
        + bp_ref[...]
    )


def _make_agg(v_pad, kc0, kc1):
    rps = v_pad // NS
    kcmax = max(kc0, kc1)

    mesh = plsc.VectorSubcoreMesh(core_axis_name="c", subcore_axis_name="s")

    @functools.partial(
        pl.kernel,
        mesh=mesh,
        out_type=[
            jax.ShapeDtypeStruct((NC, v_pad, 128), jnp.float32),
            jax.ShapeDtypeStruct((NC, v_pad), jnp.float32),
        ],
        scratch_types=[
            pltpu.VMEM((kcmax, CHUNK), jnp.int32),
            [pltpu.VMEM((CHUNK,), jnp.int32) for _ in range(NBUF)],
            [pltpu.VMEM((CHUNK,), jnp.float32) for _ in range(NBUF)],
            [pltpu.VMEM((CHUNK, 128), jnp.float32) for _ in range(NBUF)],
            pltpu.VMEM((640,), jnp.float32),
            pltpu.VMEM_SHARED((v_pad, 128), jnp.float32),
            pltpu.VMEM_SHARED((v_pad,), jnp.float32),
            [pltpu.SemaphoreType.DMA for _ in range(NBUF)],
            [pltpu.SemaphoreType.DMA for _ in range(NBUF)],
        ],
    )
    def agg(hv_hbm, src_hbm, dst_hbm, ex_hbm, cm_out, s_out,
            src_v, dst_b, ex_b, rows_b, zs_v, cm_sh, s_sh, isem, osem):
        c = lax.axis_index("c")
        s = lax.axis_index("s")
        wid = s * NC + c
        nch = jnp.where(c == 0, kc0, kc1)

        zv = jnp.zeros((L,), jnp.float32)

        def zrow(i, carry):
            for j in range(128 // L):
                rows_b[0][i, pl.ds(j * L, L)] = zv
            return carry

        lax.fori_loop(0, CHUNK, zrow, 0)

        def zsrow(i, carry):
            zs_v[pl.ds(i * L, L)] = zv
            return carry

        lax.fori_loop(0, 640 // L, zsrow, 0)

        nfull = rps // CHUNK
        for i in range(nfull):
            pltpu.sync_copy(rows_b[0],
                            cm_sh.at[pl.ds(s * rps + i * CHUNK, CHUNK)])
        rem = rps - nfull * CHUNK
        if rem:
            pltpu.sync_copy(rows_b[0].at[pl.ds(0, rem)],
                            cm_sh.at[pl.ds(s * rps + nfull * CHUNK, rem)])
        pltpu.sync_copy(zs_v.at[pl.ds(0, rps)] if rps != 640 else zs_v,
                        s_sh.at[pl.ds(s * rps, rps)])

        pltpu.sync_copy(src_hbm.at[wid], src_v)

        plsc.subcore_barrier()

        def issue_in(q, b):
            pltpu.async_copy(hv_hbm.at[src_v.at[q]], rows_b[b], isem[b])
            pltpu.async_copy(dst_hbm.at[wid, q], dst_b[b], isem[b])
            pltpu.async_copy(ex_hbm.at[wid, q], ex_b[b], isem[b])

        def wait_in(q, b):
            pltpu.make_async_copy(hv_hbm.at[src_v.at[q]], rows_b[b],
                                  isem[b]).wait()
            pltpu.make_async_copy(dst_hbm.at[wid, q], dst_b[b],
                                  isem[b]).wait()
            pltpu.make_async_copy(ex_hbm.at[wid, q], ex_b[b],
                                  isem[b]).wait()

        def issue_out(b):
            pltpu.async_copy(rows_b[b], cm_sh.at[dst_b[b]], osem[b], add=True)
            pltpu.async_copy(ex_b[b], s_sh.at[dst_b[b]], osem[b], add=True)

        def wait_out(b):
            pltpu.make_async_copy(rows_b[b], cm_sh.at[dst_b[b]],
                                  osem[b]).wait()
            pltpu.make_async_copy(ex_b[b], s_sh.at[dst_b[b]], osem[b]).wait()

        issue_in(0, 0)

        nkk = nch // NBUF
        kk_last = nkk - 1

        def outer(kk, carry):
            k0 = kk * NBUF
            for b in range(NBUF):
                k = k0 + b
                ob = 1 - b
                wait_in(k, b)

                if b == 0:
                    @pl.when(kk > 0)
                    def _():
                        wait_out(ob)

                    issue_in(k + 1, ob)
                else:
                    @pl.when(kk < kk_last)
                    def _():
                        wait_out(ob)
                        issue_in(k + 1, ob)

                def scale_g(g, carry2, _b=b):
                    gb = g * L
                    evec = ex_b[_b][pl.ds(gb, L)]
                    for i in range(L):
                        e = evec[i]
                        for jj in range(128 // L):
                            sl = pl.ds(jj * L, L)
                            rows_b[_b][gb + i, sl] = rows_b[_b][gb + i, sl] * e
                    return carry2

                lax.fori_loop(0, CHUNK // L, scale_g, 0)

                issue_out(b)
            return carry

        lax.fori_loop(0, nkk, outer, 0)

        for b in range(NBUF):
            wait_out(b)

        plsc.subcore_barrier()

        pltpu.sync_copy(cm_sh.at[pl.ds(s * rps, rps)],
                        cm_out.at[c].at[pl.ds(s * rps, rps)])
        pltpu.sync_copy(s_sh.at[pl.ds(s * rps, rps)],
                        s_out.at[c].at[pl.ds(s * rps, rps)])

    return agg


def _gru_body(cm_ref, s_ref, nf_ref, wih_ref, bih_ref, whh_ref, bhh_ref, o_ref):
    cm = cm_ref[0] + cm_ref[1]
    sv = s_ref[0] + s_ref[1]
    pos = sv > 0.0
    ctx = jnp.where(pos, cm / jnp.where(pos, sv, 1.0), 0.0)
    ctx = jnp.where(ctx > 0.0, ctx, jnp.exp(ctx) - 1.0)
    nf = nf_ref[...]
    gi = jnp.dot(ctx, wih_ref[...], preferred_element_type=jnp.float32) + bih_ref[...]
    gh = jnp.dot(nf, whh_ref[...], preferred_element_type=jnp.float32) + bhh_ref[...]
    d = o_ref.shape[1]
    r = jax.nn.sigmoid(gi[:, :d] + gh[:, :d])
    z = jax.nn.sigmoid(gi[:, d:2 * d] + gh[:, d:2 * d])
    n = jnp.tanh(gi[:, 2 * d:] + r * gh[:, 2 * d:])
    o_ref[...] = jnp.maximum((1.0 - z) * n + z * nf, 0.0)


def kernel(edge_index, edge_logits, node_feats, W_proj, b_proj, W_ih, b_ih,
           W_hh, b_hh):
    V, D = node_feats.shape
    H = W_proj.shape[0]
    E = edge_index.shape[1]

    ew = NS * CHUNK * NBUF
    e_pad = -(-E // ew) * ew
    kcol = e_pad // (NS * CHUNK)
    kc0 = min(kcol - NBUF, int(round(kcol * SPLIT0 / NBUF)) * NBUF)
    kc1 = kcol - kc0
    kcmax = max(kc0, kc1)

    rps = -(-V // (NS * 128)) * 128
    v_pad = NS * rps

    def to_blocks(full, fill):
        n0 = NS * kc0 * CHUNK
        c0 = full[:n0].reshape(NS, kc0, CHUNK)
        c1 = full[n0:].reshape(NS, kc1, CHUNK)
        blk = jnp.full((NS, NC, kcmax, CHUNK), fill, full.dtype)
        blk = blk.at[:, 0, :kc0].set(c0).at[:, 1, :kc1].set(c1)
        return blk.reshape(NW, kcmax, CHUNK)

    def pad_flat(flat, fill):
        return jnp.concatenate(
            [flat, jnp.full((e_pad - E,), fill, flat.dtype)])

    src = to_blocks(pad_flat(edge_index[0].astype(jnp.int32), 0), 0)
    dst = to_blocks(pad_flat(edge_index[1].astype(jnp.int32), 0), 0)

    logits = jnp.concatenate(
        [edge_logits[:, 0], jnp.full((e_pad - E,), -jnp.inf, jnp.float32)]
    ).reshape(e_pad // 128, 128)

    ex, hv = pl.pallas_call(
        _pre_body,
        out_shape=[
            jax.ShapeDtypeStruct((e_pad // 128, 128), jnp.float32),
            jax.ShapeDtypeStruct((V, H), jnp.float32),
        ],
    )(logits, node_feats, W_proj.T, b_proj.reshape(1, H))

    ex_blk = to_blocks(ex.reshape(e_pad), 0.0)

    agg = _make_agg(v_pad, kc0, kc1)
    cm, sacc = agg(hv, src, dst, ex_blk)

    nf_pad = jnp.concatenate(
        [node_feats, jnp.zeros((v_pad - V, D), jnp.float32)]
    )
    bv = rps
    grid = v_pad // bv
    out = pl.pallas_call(
        _gru_body,
        grid=(grid,),
        in_specs=[
            pl.BlockSpec((NC, bv, H), lambda i: (0, i, 0)),
            pl.BlockSpec((NC, bv, 1), lambda i: (0, i, 0)),
            pl.BlockSpec((bv, D), lambda i: (i, 0)),
            pl.BlockSpec((H, 3 * D), lambda i: (0, 0)),
            pl.BlockSpec((1, 3 * D), lambda i: (0, 0)),
            pl.BlockSpec((D, 3 * D), lambda i: (0, 0)),
            pl.BlockSpec((1, 3 * D), lambda i: (0, 0)),
        ],
        out_specs=pl.BlockSpec((bv, D), lambda i: (i, 0)),
        out_shape=jax.ShapeDtypeStruct((v_pad, D), jnp.float32),
    )(cm, sacc.reshape(NC, v_pad, 1), nf_pad, W_ih.T, b_ih.reshape(1, 3 * D),
      W_hh.T, b_hh.reshape(1, 3 * D))

    return out[:V]


# speedup vs baseline: 1.9675x; 1.0501x over previous
import functools

import jax
import jax.numpy as jnp
from jax import lax
from jax.experimental import pallas as pl
from jax.experimental.pallas import tpu as pltpu
from jax.experimental.pallas import tpu_sc as plsc

NC = 2
NS = 16
L = 16
NW = NC * NS
CHUNK = 64
NBUF = 2
SPLIT0 = 0.62


def _pre_body(logits_ref, nf_ref, wp_ref, bp_ref, ex_ref, hv_ref):
    l = logits_ref[...]
    m = jnp.max(l)
    ex_ref[...] = jnp.exp(l - m)
    hv_ref[...] = (
        jnp.dot(nf_ref[...], wp_ref[...], preferred_element_type=jnp.float32)
        + bp_ref[...]
    )


def _make_agg(v_pad, kc0, kc1):
    rps = v_pad // NS
    kcmax = max(kc0, kc1)

    mesh = plsc.VectorSubcoreMesh(core_axis_name="c", subcore_axis_name="s")

    @functools.partial(
        pl.kernel,
        mesh=mesh,
        out_type=[
            jax.ShapeDtypeStruct((NC, v_pad, 128), jnp.float32),
            jax.ShapeDtypeStruct((NC, v_pad), jnp.float32),
        ],
        scratch_types=[
            pltpu.VMEM((kcmax, CHUNK), jnp.int32),
            [pltpu.VMEM((CHUNK,), jnp.int32) for _ in range(NBUF)],
            [pltpu.VMEM((CHUNK,), jnp.float32) for _ in range(NBUF)],
            [pltpu.VMEM((CHUNK, 128), jnp.float32) for _ in range(NBUF)],
            pltpu.VMEM((640,), jnp.float32),
            pltpu.VMEM_SHARED((v_pad, 128), jnp.float32),
            pltpu.VMEM_SHARED((v_pad,), jnp.float32),
            [pltpu.SemaphoreType.DMA for _ in range(NBUF)],
            [pltpu.SemaphoreType.DMA for _ in range(NBUF)],
        ],
    )
    def agg(hv_hbm, src_hbm, dst_hbm, ex_hbm, cm_out, s_out,
            src_v, dst_b, ex_b, rows_b, zs_v, cm_sh, s_sh, isem, osem):
        c = lax.axis_index("c")
        s = lax.axis_index("s")
        wid = s * NC + c
        nch = jnp.where(c == 0, kc0, kc1)

        zv = jnp.zeros((L,), jnp.float32)

        def zrow(i, carry):
            for j in range(128 // L):
                rows_b[0][i, pl.ds(j * L, L)] = zv
            return carry

        lax.fori_loop(0, CHUNK, zrow, 0)

        def zsrow(i, carry):
            zs_v[pl.ds(i * L, L)] = zv
            return carry

        lax.fori_loop(0, 640 // L, zsrow, 0)

        nfull = rps // CHUNK
        for i in range(nfull):
            pltpu.sync_copy(rows_b[0],
                            cm_sh.at[pl.ds(s * rps + i * CHUNK, CHUNK)])
        rem = rps - nfull * CHUNK
        if rem:
            pltpu.sync_copy(rows_b[0].at[pl.ds(0, rem)],
                            cm_sh.at[pl.ds(s * rps + nfull * CHUNK, rem)])
        pltpu.sync_copy(zs_v.at[pl.ds(0, rps)] if rps != 640 else zs_v,
                        s_sh.at[pl.ds(s * rps, rps)])

        pltpu.sync_copy(src_hbm.at[wid], src_v)

        plsc.subcore_barrier()

        def issue_in(q, b):
            pltpu.async_copy(hv_hbm.at[src_v.at[q]], rows_b[b], isem[b])
            pltpu.async_copy(dst_hbm.at[wid, q], dst_b[b], isem[b])
            pltpu.async_copy(ex_hbm.at[wid, q], ex_b[b], isem[b])

        def wait_in(q, b):
            pltpu.make_async_copy(hv_hbm.at[src_v.at[q]], rows_b[b],
                                  isem[b]).wait()
            pltpu.make_async_copy(dst_hbm.at[wid, q], dst_b[b],
                                  isem[b]).wait()
            pltpu.make_async_copy(ex_hbm.at[wid, q], ex_b[b],
                                  isem[b]).wait()

        def issue_out(b):
            pltpu.async_copy(rows_b[b], cm_sh.at[dst_b[b]], osem[b], add=True)
            pltpu.async_copy(ex_b[b], s_sh.at[dst_b[b]], osem[b], add=True)

        def wait_out(b):
            pltpu.make_async_copy(rows_b[b], cm_sh.at[dst_b[b]],
                                  osem[b]).wait()
            pltpu.make_async_copy(ex_b[b], s_sh.at[dst_b[b]], osem[b]).wait()

        issue_in(0, 0)

        nkk = nch // NBUF
        kk_last = nkk - 1

        def outer(kk, carry):
            k0 = kk * NBUF
            for b in range(NBUF):
                k = k0 + b
                ob = 1 - b
                wait_in(k, b)

                if b == 0:
                    @pl.when(kk > 0)
                    def _():
                        wait_out(ob)

                    issue_in(k + 1, ob)
                else:
                    @pl.when(kk < kk_last)
                    def _():
                        wait_out(ob)
                        issue_in(k + 1, ob)

                def scale_g(g, carry2, _b=b):
                    gb = g * L
                    evec = ex_b[_b][pl.ds(gb, L)]
                    for i in range(L):
                        e = evec[i]
                        for jj in range(128 // L):
                            sl = pl.ds(jj * L, L)
                            rows_b[_b][gb + i, sl] = rows_b[_b][gb + i, sl] * e
                    return carry2

                lax.fori_loop(0, CHUNK // L, scale_g, 0)

                issue_out(b)
            return carry

        lax.fori_loop(0, nkk, outer, 0)

        for b in range(NBUF):
            wait_out(b)

        plsc.subcore_barrier()

        pltpu.sync_copy(cm_sh.at[pl.ds(s * rps, rps)],
                        cm_out.at[c].at[pl.ds(s * rps, rps)])
        pltpu.sync_copy(s_sh.at[pl.ds(s * rps, rps)],
                        s_out.at[c].at[pl.ds(s * rps, rps)])

    return agg


def _gru_body(cm_ref, s_ref, nf_ref, wih_ref, bih_ref, whh_ref, bhh_ref, o_ref):
    cm = cm_ref[0] + cm_ref[1]
    sv = s_ref[0] + s_ref[1]
    pos = sv > 0.0
    ctx = jnp.where(pos, cm / jnp.where(pos, sv, 1.0), 0.0)
    ctx = jnp.where(ctx > 0.0, ctx, jnp.exp(ctx) - 1.0)
    nf = nf_ref[...]
    gi = jnp.dot(ctx, wih_ref[...], preferred_element_type=jnp.float32) + bih_ref[...]
    gh = jnp.dot(nf, whh_ref[...], preferred_element_type=jnp.float32) + bhh_ref[...]
    d = o_ref.shape[1]
    r = jax.nn.sigmoid(gi[:, :d] + gh[:, :d])
    z = jax.nn.sigmoid(gi[:, d:2 * d] + gh[:, d:2 * d])
    n = jnp.tanh(gi[:, 2 * d:] + r * gh[:, 2 * d:])
    o_ref[...] = jnp.maximum((1.0 - z) * n + z * nf, 0.0)


def kernel(edge_index, edge_logits, node_feats, W_proj, b_proj, W_ih, b_ih,
           W_hh, b_hh):
    V, D = node_feats.shape
    H = W_proj.shape[0]
    E = edge_index.shape[1]

    ew = NS * CHUNK * NBUF
    e_pad = -(-E // ew) * ew
    kcol = e_pad // (NS * CHUNK)
    kc0 = min(kcol - NBUF, int(round(kcol * SPLIT0 / NBUF)) * NBUF)
    kc1 = kcol - kc0
    kcmax = max(kc0, kc1)

    rps = -(-V // (NS * 128)) * 128
    v_pad = NS * rps

    def to_blocks(full, fill):
        n0 = NS * kc0 * CHUNK
        c0 = full[:n0].reshape(NS, kc0, CHUNK)
        c1 = full[n0:].reshape(NS, kc1, CHUNK)
        blk = jnp.full((NS, NC, kcmax, CHUNK), fill, full.dtype)
        blk = blk.at[:, 0, :kc0].set(c0).at[:, 1, :kc1].set(c1)
        return blk.reshape(NW, kcmax, CHUNK)

    def pad_flat(flat, fill):
        return jnp.concatenate(
            [flat, jnp.full((e_pad - E,), fill, flat.dtype)])

    src = to_blocks(pad_flat(edge_index[0].astype(jnp.int32), 0), 0)
    dst = to_blocks(pad_flat(edge_index[1].astype(jnp.int32), 0), 0)

    logits = jnp.concatenate(
        [edge_logits[:, 0], jnp.full((e_pad - E,), -jnp.inf, jnp.float32)]
    ).reshape(e_pad // 128, 128)

    ex, hv = pl.pallas_call(
        _pre_body,
        out_shape=[
            jax.ShapeDtypeStruct((e_pad // 128, 128), jnp.float32),
            jax.ShapeDtypeStruct((V, H), jnp.float32),
        ],
    )(logits, node_feats, W_proj.T, b_proj.reshape(1, H))

    ex_blk = to_blocks(ex.reshape(e_pad), 0.0)

    agg = _make_agg(v_pad, kc0, kc1)
    cm, sacc = agg(hv, src, dst, ex_blk)

    nf_pad = jnp.concatenate(
        [node_feats, jnp.zeros((v_pad - V, D), jnp.float32)]
    )
    bv = rps
    grid = v_pad // bv
    out = pl.pallas_call(
        _gru_body,
        grid=(grid,),
        in_specs=[
            pl.BlockSpec((NC, bv, H), lambda i: (0, i, 0)),
            pl.BlockSpec((NC, bv, 1), lambda i: (0, i, 0)),
            pl.BlockSpec((bv, D), lambda i: (i, 0)),
            pl.BlockSpec((H, 3 * D), lambda i: (0, 0)),
            pl.BlockSpec((1, 3 * D), lambda i: (0, 0)),
            pl.BlockSpec((D, 3 * D), lambda i: (0, 0)),
            pl.BlockSpec((1, 3 * D), lambda i: (0, 0)),
        ],
        out_specs=pl.BlockSpec((bv, D), lambda i: (i, 0)),
        out_shape=jax.ShapeDtypeStruct((v_pad, D), jnp.float32),
    )(cm, sacc.reshape(NC, v_pad, 1), nf_pad, W_ih.T, b_ih.reshape(1, 3 * D),
      W_hh.T, b_hh.reshape(1, 3 * D))

    return out[:V]
